# baseline (speedup 1.0000x reference)
import jax
import jax.numpy as jnp
from jax.experimental import pallas as pl
from jax.experimental.pallas import tpu as pltpu


def _dma_kernel(f_ref, out_hbm, q4, sem):
    f = f_ref[...]
    for p in range(8):
        if p == 0:
            plane = f[0:520, :]
        else:
            plane = jnp.concatenate(
                [f[0:520, 16 * p:], f[1:521, : 16 * p]], axis=1
            )
        r = 7 - p
        for e in range(8):
            q4[e, r] = jax.lax.slice(plane, (7 - e, 0), (519 - e, 128))

    window = 16
    copies = []
    for c in range(32):
        a = 248 - 8 * c
        for e in range(8):
            k = len(copies)
            copies.append(
                pltpu.make_async_copy(
                    q4.at[e, :, pl.ds(a, 256), :],
                    out_hbm.at[c, e],
                    sem.at[k % window],
                )
            )
    for k, cp in enumerate(copies):
        cp.start()
        if k >= window - 1:
            copies[k - (window - 1)].wait()
    for cp in copies[-(window - 1):]:
        cp.wait()


def kernel(inputs, rel_embeddings):
    del inputs
    flat = jnp.flip(rel_embeddings, axis=0).reshape(-1)
    f2d = jnp.concatenate([flat, jnp.zeros((1168,), flat.dtype)]).reshape(521, 128)
    out = pl.pallas_call(
        _dma_kernel,
        in_specs=[pl.BlockSpec(memory_space=pltpu.MemorySpace.VMEM)],
        out_specs=pl.BlockSpec(memory_space=pl.ANY),
        out_shape=jax.ShapeDtypeStruct((32, 8, 8, 256, 128), jnp.float32),
        scratch_shapes=[
            pltpu.VMEM((8, 8, 512, 128), jnp.float32),
            pltpu.SemaphoreType.DMA((16,)),
        ],
    )(f2d)
    return out.reshape(2048, 2048, 16)


# speedup vs baseline: 1.0001x; 1.0001x over previous
import jax
import jax.numpy as jnp
from jax.experimental import pallas as pl
from jax.experimental.pallas import tpu as pltpu


def _dma_kernel(f_ref, out_hbm, q4, sem):
    f = f_ref[...]
    for p in range(8):
        if p == 0:
            plane = f[0:520, :]
        else:
            plane = jnp.concatenate(
                [f[0:520, 16 * p:], f[1:521, : 16 * p]], axis=1
            )
        r = 7 - p
        for e in range(8):
            q4[e, r] = jax.lax.slice(plane, (7 - e, 0), (519 - e, 128))

    window = 16

    def _block_copy(k):
        c = jax.lax.div(k, 8)
        e = jax.lax.rem(k, 8)
        a = 248 - 8 * c
        return pltpu.make_async_copy(
            q4.at[e, :, pl.ds(a, 256), :],
            out_hbm.at[c, e],
            sem.at[jax.lax.rem(k, window)],
        )

    def body(k, _):
        _block_copy(k).start()

        @pl.when(k >= window - 1)
        def _():
            _block_copy(k - (window - 1)).wait()

        return 0

    jax.lax.fori_loop(0, 256, body, 0)

    def tail(k, _):
        _block_copy(k).wait()
        return 0

    jax.lax.fori_loop(256 - (window - 1), 256, tail, 0)


def kernel(inputs, rel_embeddings):
    del inputs
    flat = jnp.flip(rel_embeddings, axis=0).reshape(-1)
    f2d = jnp.concatenate([flat, jnp.zeros((1168,), flat.dtype)]).reshape(521, 128)
    out = pl.pallas_call(
        _dma_kernel,
        in_specs=[pl.BlockSpec(memory_space=pltpu.MemorySpace.VMEM)],
        out_specs=pl.BlockSpec(memory_space=pl.ANY),
        out_shape=jax.ShapeDtypeStruct((32, 8, 8, 256, 128), jnp.float32),
        scratch_shapes=[
            pltpu.VMEM((8, 8, 512, 128), jnp.float32),
            pltpu.SemaphoreType.DMA((16,)),
        ],
    )(f2d)
    return out.reshape(2048, 2048, 16)


# speedup vs baseline: 3.0648x; 3.0644x over previous
import jax
import jax.numpy as jnp
from jax.experimental import pallas as pl
from jax.experimental.pallas import tpu as pltpu

_NTHREADS = 2


def _dma_kernel(f_ref, out_hbm, q, sem):
    f = f_ref[...]
    for p in range(8):
        if p == 0:
            plane = f[0:512, :]
        else:
            plane = jnp.concatenate(
                [f[0:512, 16 * p:], f[1:513, : 16 * p]], axis=1
            )
        q[7 - p] = plane

    copies = [
        pltpu.make_async_copy(
            q.at[:, pl.ds(255 - b, 256), :],
            out_hbm.at[pl.ds(8 * b, 8), :, :],
            sem,
        )
        for b in range(256)
    ]
    for b, cp in enumerate(copies):
        cp.start(priority=b % _NTHREADS)
    for cp in copies:
        cp.wait()


def kernel(inputs, rel_embeddings):
    del inputs
    flat = jnp.flip(rel_embeddings, axis=0).reshape(-1)
    f2d = jnp.concatenate([flat, jnp.zeros((1168,), flat.dtype)]).reshape(521, 128)
    out = pl.pallas_call(
        _dma_kernel,
        in_specs=[pl.BlockSpec(memory_space=pltpu.MemorySpace.VMEM)],
        out_specs=pl.BlockSpec(memory_space=pl.ANY),
        out_shape=jax.ShapeDtypeStruct((2048, 256, 128), jnp.float32),
        scratch_shapes=[
            pltpu.VMEM((8, 512, 128), jnp.float32),
            pltpu.SemaphoreType.DMA,
        ],
    )(f2d)
    return out.reshape(2048, 2048, 16)
